# baseline (speedup 1.0000x reference)
import functools

import jax
import jax.numpy as jnp
from jax import lax
from jax.experimental import pallas as pl
from jax.experimental.pallas import tpu as pltpu
from jax.experimental.pallas import tpu_sc as plsc

_P = 7
_NB = _P * _P
_RPB = 16
_ROWS = _NB * _RPB
_CH = 128
_NSC = 16
_PAIR = 64
_M0 = 24
_CAP = 48
_NPAD = _NSC * _PAIR
_NST = 1088
_CHUNK_BINS = 7
_CHUNK_ROWS = _CHUNK_BINS * _RPB
_NCHUNK = _NB // _CHUNK_BINS
_OB = _NB * _CH

_LEVEL_OFF = (0, 80000, 100000, 105000)
_AREA_T = (12544.0, 50176.0, 200704.0)

_GATHER_DN = lax.GatherDimensionNumbers(
    offset_dims=(), collapsed_slice_dims=(0,), start_index_map=(0,))


def _splat16(v, i):
    idx = jnp.full((16, 1), i, jnp.int32)
    return lax.gather(v, idx, _GATHER_DN, (1,),
                      mode=lax.GatherScatterMode.PROMISE_IN_BOUNDS)


def _splat48(ref, base, i):
    outs = []
    for h in range(_CAP // 16):
        v = ref[pl.ds(base + h * 16, 16)]
        li = jnp.clip(i - h * 16, 0, 15)
        outs.append(_splat16(v, li))
    return jnp.where(i < 16, outs[0], jnp.where(i < 32, outs[1], outs[2]))


def _sc_pool(table, bt, per_img):
    mesh = plsc.VectorSubcoreMesh(core_axis_name="c", subcore_axis_name="s")

    @functools.partial(
        pl.kernel,
        out_type=(
            jax.ShapeDtypeStruct((_NPAD * _OB,), jnp.float32),
            jax.ShapeDtypeStruct((_NST,), jnp.float32),
        ),
        mesh=mesh,
        scratch_types=[
            pltpu.VMEM((4 * _CAP,), jnp.float32),
            pltpu.VMEM((5 * _CAP,), jnp.float32),
            pltpu.VMEM((2 * _CAP,), jnp.int32),
            pltpu.VMEM((_ROWS,), jnp.int32),
            pltpu.VMEM((_ROWS,), jnp.int32),
            pltpu.VMEM((_ROWS,), jnp.float32),
            pltpu.VMEM((_ROWS,), jnp.float32),
            pltpu.VMEM((_CHUNK_ROWS, _CH), jnp.float32),
            pltpu.VMEM((_CHUNK_ROWS, _CH), jnp.float32),
            pltpu.VMEM((_OB,), jnp.float32),
            pltpu.VMEM((_CAP,), jnp.float32),
            pltpu.SemaphoreType.DMA,
            pltpu.SemaphoreType.DMA,
        ],
    )
    def kfn(table_h, bt_h, out_h, masks_h,
            bt_v, prm_f, prm_i, idx_0, idx_1, w_0, w_1, rows_a, rows_b,
            out_v, masks_v, sem_a, sem_b):
        cid = lax.axis_index("c")
        sid = lax.axis_index("s")
        base_box = sid * _PAIR + jnp.where(cid == 0, 0, _M0)
        n_pairs = jnp.where(cid == 0, _M0 // 2, (_PAIR - _M0) // 2)

        for c in range(4):
            pltpu.sync_copy(bt_h.at[pl.ds(c * _NST + base_box, _CAP)],
                            bt_v.at[pl.ds(c * _CAP, _CAP)])

        lane = lax.iota(jnp.int32, 16)

        for h in range(_CAP // 16):
            sl = pl.ds(h * 16, 16)
            x1 = bt_v[pl.ds(0 * _CAP + h * 16, 16)]
            y1 = bt_v[pl.ds(1 * _CAP + h * 16, 16)]
            x2 = bt_v[pl.ds(2 * _CAP + h * 16, 16)]
            y2 = bt_v[pl.ds(3 * _CAP + h * 16, 16)]
            widths = x2 - x1 + 1.0
            heights = y2 - y1 + 1.0
            ctr_x = x1 + 0.5 * widths
            ctr_y = y1 + 0.5 * heights
            ew = widths * 1.5
            eh = heights * 1.5
            ex1 = ctr_x - 0.5 * ew
            ey1 = ctr_y - 0.5 * eh
            ex2 = ctr_x + 0.5 * ew - 1.0
            ey2 = ctr_y + 0.5 * eh - 1.0
            area = widths * heights
            lvl = (jnp.where(area >= _AREA_T[0], 1, 0)
                   + jnp.where(area >= _AREA_T[1], 1, 0)
                   + jnp.where(area >= _AREA_T[2], 1, 0))
            maskv = 0.25 * (lvl + 1).astype(jnp.float32)
            wi = lax.shift_right_logical(jnp.full((16,), 200, jnp.int32), lvl)
            scale = jnp.where(
                lvl == 0, 0.25,
                jnp.where(lvl == 1, 0.125,
                          jnp.where(lvl == 2, 0.0625, 0.03125)))
            off = jnp.where(
                lvl == 0, _LEVEL_OFF[0],
                jnp.where(lvl == 1, _LEVEL_OFF[1],
                          jnp.where(lvl == 2, _LEVEL_OFF[2], _LEVEL_OFF[3])))
            gidx = base_box + h * 16 + lane
            img = jnp.where(gidx >= per_img, 1, 0)
            basei = off + img * wi * wi
            x1s = ex1 * scale
            y1s = ey1 * scale
            x2s = ex2 * scale
            y2s = ey2 * scale
            roi_w = jnp.maximum(x2s - x1s, 1.0)
            roi_h = jnp.maximum(y2s - y1s, 1.0)
            prm_f[pl.ds(0 * _CAP + h * 16, 16)] = x1s
            prm_f[pl.ds(1 * _CAP + h * 16, 16)] = y1s
            prm_f[pl.ds(2 * _CAP + h * 16, 16)] = roi_w / 7.0
            prm_f[pl.ds(3 * _CAP + h * 16, 16)] = roi_h / 7.0
            prm_f[pl.ds(4 * _CAP + h * 16, 16)] = wi.astype(jnp.float32)
            prm_i[pl.ds(0 * _CAP + h * 16, 16)] = basei
            prm_i[pl.ds(1 * _CAP + h * 16, 16)] = wi
            masks_v[sl] = maskv
        pltpu.sync_copy(masks_v, masks_h.at[pl.ds(base_box, _CAP)])

        p = lax.shift_right_logical(lane, 2)
        t = lane & 3
        gy_off = ((lax.shift_right_logical(p, 1)).astype(jnp.float32) + 0.5) * 0.5
        gx_off = ((p & 1).astype(jnp.float32) + 0.5) * 0.5
        t_hi = lax.shift_right_logical(t, 1) == 1
        t_lo = (t & 1) == 1

        idx_bufs = (idx_0, idx_1)
        w_bufs = (w_0, w_1)
        rows_bufs = (rows_a, rows_b)
        sems = (sem_a, sem_b)

        def _buf_par(slot, k):
            return (k + slot) & 1

        def phase1(i, slot):
            idx_v = idx_bufs[slot]
            w_v = w_bufs[slot]
            x1s = _splat48(prm_f, 0 * _CAP, i)
            y1s = _splat48(prm_f, 1 * _CAP, i)
            binw = _splat48(prm_f, 2 * _CAP, i)
            binh = _splat48(prm_f, 3 * _CAP, i)
            wf = _splat48(prm_f, 4 * _CAP, i)
            basei = _splat48(prm_i, 0 * _CAP, i)
            wi = _splat48(prm_i, 1 * _CAP, i)

            def by_body(by, carry_y):
                byf = by.astype(jnp.float32)
                ys = y1s + (byf + gy_off) * binh
                valid_y = (ys > -1.0) & (ys < wf)
                y = jnp.maximum(ys, 0.0)
                y_low = jnp.minimum(y.astype(jnp.int32), wi - 1)
                y_high = jnp.minimum(y_low + 1, wi - 1)
                ly = jnp.where(y_low >= wi - 1, 0.0,
                               y - y_low.astype(jnp.float32))
                wy = jnp.where(t_hi, ly, 1.0 - ly)
                ty = jnp.where(t_hi, y_high, y_low)

                def bx_body(bx, carry_x):
                    bxf = bx.astype(jnp.float32)
                    xs = x1s + (bxf + gx_off) * binw
                    valid = valid_y & (xs > -1.0) & (xs < wf)
                    x = jnp.maximum(xs, 0.0)
                    x_low = jnp.minimum(x.astype(jnp.int32), wi - 1)
                    x_high = jnp.minimum(x_low + 1, wi - 1)
                    lx = jnp.where(x_low >= wi - 1, 0.0,
                                   x - x_low.astype(jnp.float32))
                    wx = jnp.where(t_lo, lx, 1.0 - lx)
                    tx = jnp.where(t_lo, x_high, x_low)
                    w = jnp.where(valid, wy * wx, 0.0)
                    row = basei + ty * wi + tx
                    b = by * _P + bx
                    idx_v[pl.ds(b * _RPB, _RPB)] = row
                    w_v[pl.ds(b * _RPB, _RPB)] = w
                    return carry_x

                return lax.fori_loop(0, _P, bx_body, carry_y)

            lax.fori_loop(0, _P, by_body, 0)

        def issue(slot, k):
            bi = _buf_par(slot, k)
            rs = pl.ds(k * _CHUNK_ROWS, _CHUNK_ROWS)
            cp = pltpu.make_async_copy(
                table_h.at[idx_bufs[slot].at[rs]], rows_bufs[bi], sems[bi])
            cp.start()
            return cp

        def acc_chunk(slot, k):
            rbuf = rows_bufs[_buf_par(slot, k)]
            w_v = w_bufs[slot]

            def acc_body(j, carry_b, _k=k, _rbuf=rbuf, _wv=w_v):
                b = _k * _CHUNK_BINS + j
                w16 = _wv[pl.ds(b * _RPB, _RPB)]
                acc = [jnp.zeros((16,), jnp.float32)
                       for _ in range(_CH // 16)]
                for jj in range(_RPB):
                    wj = _splat16(w16, jj)
                    for c in range(_CH // 16):
                        acc[c] = acc[c] + wj * _rbuf[j * _RPB + jj,
                                                     pl.ds(c * 16, 16)]
                for c in range(_CH // 16):
                    out_v[pl.ds(b * _CH + c * 16, 16)] = acc[c] * 0.25
                return carry_b

            lax.fori_loop(0, _CHUNK_BINS, acc_body, 0)

        def out_copy(i):
            g = base_box + i
            pltpu.sync_copy(out_v, out_h.at[pl.ds(g * _OB, _OB)])

        def do_pair(j2, carry):
            iA = 2 * j2
            iB = iA + 1
            phase1(iA, 0)
            cpA = {0: issue(0, 0), 1: issue(0, 1)}
            for k in range(_NCHUNK - 2):
                cpA[k].wait()
                acc_chunk(0, k)
                cpA[k + 2] = issue(0, k + 2)
            phase1(iB, 1)
            cpA[_NCHUNK - 2].wait()
            acc_chunk(0, _NCHUNK - 2)
            cpB = {0: issue(1, 0)}
            cpA[_NCHUNK - 1].wait()
            acc_chunk(0, _NCHUNK - 1)
            cpB[1] = issue(1, 1)
            out_copy(iA)
            for k in range(_NCHUNK - 2):
                cpB[k].wait()
                acc_chunk(1, k)
                cpB[k + 2] = issue(1, k + 2)
            cpB[_NCHUNK - 2].wait()
            acc_chunk(1, _NCHUNK - 2)
            cpB[_NCHUNK - 1].wait()
            acc_chunk(1, _NCHUNK - 1)
            out_copy(iB)
            return carry

        lax.fori_loop(0, n_pairs, do_pair, 0)

    return kfn(table, bt)


def kernel(feat0, feat1, feat2, feat3, boxes):
    n = boxes.shape[0]
    tabs = []
    for f in (feat0, feat1, feat2, feat3):
        b, c, hh, ww = f.shape
        tabs.append(jnp.transpose(f, (0, 2, 3, 1)).reshape(b * hh * ww, c))
    table = jnp.concatenate(tabs, axis=0)
    bt = jnp.zeros((4, _NST), jnp.float32).at[:, :n].set(boxes.T).reshape(-1)
    out, masks = _sc_pool(table, bt, n // 2)
    res = out.reshape(_NPAD, _P, _P, _CH)[:n].transpose(0, 3, 1, 2)
    return res, masks[:n]


# speedup vs baseline: 1.2014x; 1.2014x over previous
import functools

import jax
import jax.numpy as jnp
from jax import lax
from jax.experimental import pallas as pl
from jax.experimental.pallas import tpu as pltpu
from jax.experimental.pallas import tpu_sc as plsc

_P = 7
_NB = _P * _P
_RPB = 16
_ROWS = _NB * _RPB
_CH = 128
_NSC = 16
_PAIR = 64
_M0 = 40
_CAP = 48
_NPAD = _NSC * _PAIR
_NST = 1088
_CHUNK_BINS = 7
_CHUNK_ROWS = _CHUNK_BINS * _RPB
_NCHUNK = _NB // _CHUNK_BINS
_OB = _NB * _CH

_LEVEL_OFF = (0, 80000, 100000, 105000)
_AREA_T = (12544.0, 50176.0, 200704.0)

_GATHER_DN = lax.GatherDimensionNumbers(
    offset_dims=(), collapsed_slice_dims=(0,), start_index_map=(0,))


def _splat16(v, i):
    idx = jnp.full((16, 1), i, jnp.int32)
    return lax.gather(v, idx, _GATHER_DN, (1,),
                      mode=lax.GatherScatterMode.PROMISE_IN_BOUNDS)


def _splat48(ref, base, i):
    outs = []
    for h in range(_CAP // 16):
        v = ref[pl.ds(base + h * 16, 16)]
        li = jnp.clip(i - h * 16, 0, 15)
        outs.append(_splat16(v, li))
    return jnp.where(i < 16, outs[0], jnp.where(i < 32, outs[1], outs[2]))


def _sc_pool(table, bt, per_img):
    mesh = plsc.VectorSubcoreMesh(core_axis_name="c", subcore_axis_name="s")

    @functools.partial(
        pl.kernel,
        out_type=(
            jax.ShapeDtypeStruct((_NPAD * _OB,), jnp.float32),
            jax.ShapeDtypeStruct((_NST,), jnp.float32),
        ),
        mesh=mesh,
        scratch_types=[
            pltpu.VMEM((4 * _CAP,), jnp.float32),
            pltpu.VMEM((5 * _CAP,), jnp.float32),
            pltpu.VMEM((2 * _CAP,), jnp.int32),
            pltpu.VMEM((_ROWS,), jnp.int32),
            pltpu.VMEM((_ROWS,), jnp.int32),
            pltpu.VMEM((_ROWS,), jnp.float32),
            pltpu.VMEM((_ROWS,), jnp.float32),
            pltpu.VMEM((_CHUNK_ROWS, _CH), jnp.float32),
            pltpu.VMEM((_CHUNK_ROWS, _CH), jnp.float32),
            pltpu.VMEM((_OB,), jnp.float32),
            pltpu.VMEM((_CAP,), jnp.float32),
            pltpu.SemaphoreType.DMA,
            pltpu.SemaphoreType.DMA,
        ],
    )
    def kfn(table_h, bt_h, out_h, masks_h,
            bt_v, prm_f, prm_i, idx_0, idx_1, w_0, w_1, rows_a, rows_b,
            out_v, masks_v, sem_a, sem_b):
        cid = lax.axis_index("c")
        sid = lax.axis_index("s")
        base_box = sid * _PAIR + jnp.where(cid == 0, 0, _M0)
        n_pairs = jnp.where(cid == 0, _M0 // 2, (_PAIR - _M0) // 2)

        for c in range(4):
            pltpu.sync_copy(bt_h.at[pl.ds(c * _NST + base_box, _CAP)],
                            bt_v.at[pl.ds(c * _CAP, _CAP)])

        lane = lax.iota(jnp.int32, 16)

        for h in range(_CAP // 16):
            sl = pl.ds(h * 16, 16)
            x1 = bt_v[pl.ds(0 * _CAP + h * 16, 16)]
            y1 = bt_v[pl.ds(1 * _CAP + h * 16, 16)]
            x2 = bt_v[pl.ds(2 * _CAP + h * 16, 16)]
            y2 = bt_v[pl.ds(3 * _CAP + h * 16, 16)]
            widths = x2 - x1 + 1.0
            heights = y2 - y1 + 1.0
            ctr_x = x1 + 0.5 * widths
            ctr_y = y1 + 0.5 * heights
            ew = widths * 1.5
            eh = heights * 1.5
            ex1 = ctr_x - 0.5 * ew
            ey1 = ctr_y - 0.5 * eh
            ex2 = ctr_x + 0.5 * ew - 1.0
            ey2 = ctr_y + 0.5 * eh - 1.0
            area = widths * heights
            lvl = (jnp.where(area >= _AREA_T[0], 1, 0)
                   + jnp.where(area >= _AREA_T[1], 1, 0)
                   + jnp.where(area >= _AREA_T[2], 1, 0))
            maskv = 0.25 * (lvl + 1).astype(jnp.float32)
            wi = lax.shift_right_logical(jnp.full((16,), 200, jnp.int32), lvl)
            scale = jnp.where(
                lvl == 0, 0.25,
                jnp.where(lvl == 1, 0.125,
                          jnp.where(lvl == 2, 0.0625, 0.03125)))
            off = jnp.where(
                lvl == 0, _LEVEL_OFF[0],
                jnp.where(lvl == 1, _LEVEL_OFF[1],
                          jnp.where(lvl == 2, _LEVEL_OFF[2], _LEVEL_OFF[3])))
            gidx = base_box + h * 16 + lane
            img = jnp.where(gidx >= per_img, 1, 0)
            basei = off + img * wi * wi
            x1s = ex1 * scale
            y1s = ey1 * scale
            x2s = ex2 * scale
            y2s = ey2 * scale
            roi_w = jnp.maximum(x2s - x1s, 1.0)
            roi_h = jnp.maximum(y2s - y1s, 1.0)
            prm_f[pl.ds(0 * _CAP + h * 16, 16)] = x1s
            prm_f[pl.ds(1 * _CAP + h * 16, 16)] = y1s
            prm_f[pl.ds(2 * _CAP + h * 16, 16)] = roi_w / 7.0
            prm_f[pl.ds(3 * _CAP + h * 16, 16)] = roi_h / 7.0
            prm_f[pl.ds(4 * _CAP + h * 16, 16)] = wi.astype(jnp.float32)
            prm_i[pl.ds(0 * _CAP + h * 16, 16)] = basei
            prm_i[pl.ds(1 * _CAP + h * 16, 16)] = wi
            masks_v[sl] = maskv
        pltpu.sync_copy(masks_v, masks_h.at[pl.ds(base_box, _CAP)])

        p = lax.shift_right_logical(lane, 2)
        t = lane & 3
        gy_off = ((lax.shift_right_logical(p, 1)).astype(jnp.float32) + 0.5) * 0.5
        gx_off = ((p & 1).astype(jnp.float32) + 0.5) * 0.5
        t_hi = lax.shift_right_logical(t, 1) == 1
        t_lo = (t & 1) == 1

        idx_bufs = (idx_0, idx_1)
        w_bufs = (w_0, w_1)
        rows_bufs = (rows_a, rows_b)
        sems = (sem_a, sem_b)

        def _buf_par(slot, k):
            return (k + slot) & 1

        def phase1(i, slot):
            idx_v = idx_bufs[slot]
            w_v = w_bufs[slot]
            x1s = _splat48(prm_f, 0 * _CAP, i)
            y1s = _splat48(prm_f, 1 * _CAP, i)
            binw = _splat48(prm_f, 2 * _CAP, i)
            binh = _splat48(prm_f, 3 * _CAP, i)
            wf = _splat48(prm_f, 4 * _CAP, i)
            basei = _splat48(prm_i, 0 * _CAP, i)
            wi = _splat48(prm_i, 1 * _CAP, i)

            def by_body(by, carry_y):
                byf = by.astype(jnp.float32)
                ys = y1s + (byf + gy_off) * binh
                valid_y = (ys > -1.0) & (ys < wf)
                y = jnp.maximum(ys, 0.0)
                y_low = jnp.minimum(y.astype(jnp.int32), wi - 1)
                y_high = jnp.minimum(y_low + 1, wi - 1)
                ly = jnp.where(y_low >= wi - 1, 0.0,
                               y - y_low.astype(jnp.float32))
                wy = jnp.where(t_hi, ly, 1.0 - ly)
                ty = jnp.where(t_hi, y_high, y_low)

                def bx_body(bx, carry_x):
                    bxf = bx.astype(jnp.float32)
                    xs = x1s + (bxf + gx_off) * binw
                    valid = valid_y & (xs > -1.0) & (xs < wf)
                    x = jnp.maximum(xs, 0.0)
                    x_low = jnp.minimum(x.astype(jnp.int32), wi - 1)
                    x_high = jnp.minimum(x_low + 1, wi - 1)
                    lx = jnp.where(x_low >= wi - 1, 0.0,
                                   x - x_low.astype(jnp.float32))
                    wx = jnp.where(t_lo, lx, 1.0 - lx)
                    tx = jnp.where(t_lo, x_high, x_low)
                    w = jnp.where(valid, wy * wx, 0.0)
                    row = basei + ty * wi + tx
                    b = by * _P + bx
                    idx_v[pl.ds(b * _RPB, _RPB)] = row
                    w_v[pl.ds(b * _RPB, _RPB)] = w
                    return carry_x

                return lax.fori_loop(0, _P, bx_body, carry_y)

            lax.fori_loop(0, _P, by_body, 0)

        def issue(slot, k):
            bi = _buf_par(slot, k)
            rs = pl.ds(k * _CHUNK_ROWS, _CHUNK_ROWS)
            cp = pltpu.make_async_copy(
                table_h.at[idx_bufs[slot].at[rs]], rows_bufs[bi], sems[bi])
            cp.start()
            return cp

        def acc_chunk(slot, k):
            rbuf = rows_bufs[_buf_par(slot, k)]
            w_v = w_bufs[slot]

            def acc_body(j, carry_b, _k=k, _rbuf=rbuf, _wv=w_v):
                b = _k * _CHUNK_BINS + j
                w16 = _wv[pl.ds(b * _RPB, _RPB)]
                acc = [jnp.zeros((16,), jnp.float32)
                       for _ in range(_CH // 16)]
                for jj in range(_RPB):
                    wj = _splat16(w16, jj)
                    for c in range(_CH // 16):
                        acc[c] = acc[c] + wj * _rbuf[j * _RPB + jj,
                                                     pl.ds(c * 16, 16)]
                for c in range(_CH // 16):
                    out_v[pl.ds(b * _CH + c * 16, 16)] = acc[c] * 0.25
                return carry_b

            lax.fori_loop(0, _CHUNK_BINS, acc_body, 0)

        def out_copy(i):
            g = base_box + i
            pltpu.sync_copy(out_v, out_h.at[pl.ds(g * _OB, _OB)])

        def do_pair(j2, carry):
            iA = 2 * j2
            iB = iA + 1
            phase1(iA, 0)
            cpA = {0: issue(0, 0), 1: issue(0, 1)}
            for k in range(_NCHUNK - 2):
                cpA[k].wait()
                acc_chunk(0, k)
                cpA[k + 2] = issue(0, k + 2)
            phase1(iB, 1)
            cpA[_NCHUNK - 2].wait()
            acc_chunk(0, _NCHUNK - 2)
            cpB = {0: issue(1, 0)}
            cpA[_NCHUNK - 1].wait()
            acc_chunk(0, _NCHUNK - 1)
            cpB[1] = issue(1, 1)
            out_copy(iA)
            for k in range(_NCHUNK - 2):
                cpB[k].wait()
                acc_chunk(1, k)
                cpB[k + 2] = issue(1, k + 2)
            cpB[_NCHUNK - 2].wait()
            acc_chunk(1, _NCHUNK - 2)
            cpB[_NCHUNK - 1].wait()
            acc_chunk(1, _NCHUNK - 1)
            out_copy(iB)
            return carry

        lax.fori_loop(0, n_pairs, do_pair, 0)

    return kfn(table, bt)


def kernel(feat0, feat1, feat2, feat3, boxes):
    n = boxes.shape[0]
    tabs = []
    for f in (feat0, feat1, feat2, feat3):
        b, c, hh, ww = f.shape
        tabs.append(jnp.transpose(f, (0, 2, 3, 1)).reshape(b * hh * ww, c))
    table = jnp.concatenate(tabs, axis=0)
    bt = jnp.zeros((4, _NST), jnp.float32).at[:, :n].set(boxes.T).reshape(-1)
    out, masks = _sc_pool(table, bt, n // 2)
    res = out.reshape(_NPAD, _P, _P, _CH)[:n].transpose(0, 3, 1, 2)
    return res, masks[:n]
